# baseline (speedup 1.0000x reference)
import jax
import jax.numpy as jnp
from jax import lax
from jax.experimental import pallas as pl
from jax.experimental.pallas import tpu as pltpu


def _round_up(x, m):
    return (x + m - 1) // m * m


def _make_kernel(num_layers, t_chunk, b_blk, hidden, t_valid, t_total, unroll):
    L, B, H = num_layers, b_blk, hidden
    needs_mask = (t_valid != t_total)

    def _body(*refs):
        x_ref = refs[0]
        w_refs = refs[1:1 + 3 * L]
        fcw_ref, fcb_ref = refs[1 + 3 * L:3 + 3 * L]
        h0_ref, c0_ref = refs[3 + 3 * L:5 + 3 * L]
        hN_ref, cN_ref, out_ref = refs[5 + 3 * L:8 + 3 * L]
        h_scr, c_scr, y_scr, zx_scr = refs[8 + 3 * L:]

        t_idx = pl.program_id(1)
        n_t = pl.num_programs(1)

        @pl.when(t_idx == 0)
        def _init():
            h_scr[...] = h0_ref[...]
            c_scr[...] = c0_ref[...]

        for layer in range(L):
            wih_ref = w_refs[3 * layer + 0]
            whh_ref = w_refs[3 * layer + 1]
            b_ref = w_refs[3 * layer + 2]

            inp = x_ref[...] if layer == 0 else y_scr[...]
            zx_scr[...] = (jnp.dot(inp, wih_ref[...],
                                   preferred_element_type=jnp.float32)
                           + b_ref[...])

            write_y = (layer != L - 1)

            def step(t, carry, whh_ref=whh_ref, write_y=write_y):
                h, c = carry
                row = pl.multiple_of(t * B, 8)
                z = zx_scr[pl.ds(row, B), :] + jnp.dot(
                    h.astype(jnp.bfloat16), whh_ref[...],
                    preferred_element_type=jnp.float32)
                sig = jax.nn.sigmoid(z[:, 0:3 * H])
                i_g = sig[:, 0 * H:1 * H]
                f_g = sig[:, 1 * H:2 * H]
                o_g = sig[:, 2 * H:3 * H]
                g_g = jnp.tanh(z[:, 3 * H:4 * H])
                c_new = f_g * c + i_g * g_g
                h_new = o_g * jnp.tanh(c_new)
                if needs_mask:
                    valid = (t_idx * t_chunk + t) < t_valid
                    h_new = jnp.where(valid, h_new, h)
                    c_new = jnp.where(valid, c_new, c)
                if write_y:
                    y_scr[pl.ds(row, B), :] = h_new.astype(jnp.bfloat16)
                return (h_new, c_new)

            h_f, c_f = lax.fori_loop(0, t_chunk, step,
                                     (h_scr[layer], c_scr[layer]),
                                     unroll=min(unroll, t_chunk))
            h_scr[layer] = h_f
            c_scr[layer] = c_f

        @pl.when(t_idx == n_t - 1)
        def _finalize():
            hN_ref[...] = h_scr[...]
            cN_ref[...] = c_scr[...]
            out_ref[...] = (jnp.dot(h_scr[L - 1].astype(jnp.bfloat16),
                                    fcw_ref[...],
                                    preferred_element_type=jnp.float32)
                            + fcb_ref[...])

    return _body


def _impl(embedding, wih_t, whh_t, bias, fc_w_t, fc_b, input_sequence,
          state_h, state_c, *, t_chunk, unroll, single_buffered):
    L = len(wih_t)
    B, T = input_sequence.shape
    E = embedding.shape[1]
    H = state_h.shape[-1]
    V = fc_w_t.shape[1]

    B_pad = _round_up(B, 16)
    n_bblk = 2
    B_blk = B_pad // n_bblk

    for cand in (t_chunk, 32, 16, 8):
        if cand and T % cand == 0:
            t_chunk = cand
            break
    else:
        t_chunk = min(T, 32)
    t_chunk = min(t_chunk, T)
    T_pad = _round_up(T, t_chunk)
    n_chunks = T_pad // t_chunk

    def reorder(w):
        return jnp.concatenate(
            [w[:, :2 * H], w[:, 3 * H:4 * H], w[:, 2 * H:3 * H]], axis=1)

    wih = [reorder(w).astype(jnp.bfloat16) for w in wih_t]
    whh = [reorder(w).astype(jnp.bfloat16) for w in whh_t]
    bia = [reorder(b).astype(jnp.float32) for b in bias]

    V_pad = _round_up(V, 128)
    fcw = fc_w_t
    fcb = fc_b
    if V_pad != V:
        fcw = jnp.pad(fcw, ((0, 0), (0, V_pad - V)))
        fcb = jnp.pad(fcb, ((0, 0), (0, V_pad - V)))
    fcw = fcw.astype(jnp.bfloat16)
    fcb = fcb.astype(jnp.float32)

    tok = input_sequence
    if B_pad != B:
        tok = jnp.pad(tok, ((0, B_pad - B), (0, 0)))
        state_h = jnp.pad(state_h, ((0, 0), (0, B_pad - B), (0, 0)))
        state_c = jnp.pad(state_c, ((0, 0), (0, B_pad - B), (0, 0)))
    if T_pad != T:
        tok = jnp.pad(tok, ((0, 0), (0, T_pad - T)))
    tok = tok.reshape(n_bblk, B_blk, T_pad).transpose(0, 2, 1).reshape(-1)
    x2d = jnp.take(embedding.astype(jnp.bfloat16), tok, axis=0)

    def const_spec(shape, index_map):
        if single_buffered:
            return pl.BlockSpec(shape, index_map, pipeline_mode=pl.Buffered(1))
        return pl.BlockSpec(shape, index_map)

    in_specs = [pl.BlockSpec((t_chunk * B_blk, E),
                             lambda b, t: (b * n_chunks + t, 0))]
    flat_inputs = [x2d]
    for layer in range(L):
        for arr in (wih[layer], whh[layer], bia[layer]):
            nd = arr.ndim
            in_specs.append(const_spec(arr.shape, lambda b, t, nd=nd: (0,) * nd))
            flat_inputs.append(arr)
    in_specs.append(const_spec((H, V_pad), lambda b, t: (0, 0)))
    flat_inputs.append(fcw)
    in_specs.append(const_spec((1, V_pad), lambda b, t: (0, 0)))
    flat_inputs.append(fcb)
    for arr in (state_h.astype(jnp.float32), state_c.astype(jnp.float32)):
        in_specs.append(pl.BlockSpec((L, B_blk, H), lambda b, t: (0, b, 0)))
        flat_inputs.append(arr)

    out_shape = (jax.ShapeDtypeStruct((L, B_pad, H), jnp.float32),
                 jax.ShapeDtypeStruct((L, B_pad, H), jnp.float32),
                 jax.ShapeDtypeStruct((B_pad, V_pad), jnp.float32))
    out_specs = (pl.BlockSpec((L, B_blk, H), lambda b, t: (0, b, 0)),
                 pl.BlockSpec((L, B_blk, H), lambda b, t: (0, b, 0)),
                 pl.BlockSpec((B_blk, V_pad), lambda b, t: (b, 0)))

    scratch_shapes = [
        pltpu.VMEM((L, B_blk, H), jnp.float32),
        pltpu.VMEM((L, B_blk, H), jnp.float32),
        pltpu.VMEM((t_chunk * B_blk, H), jnp.bfloat16),
        pltpu.VMEM((t_chunk * B_blk, 4 * H), jnp.float32),
    ]

    body = _make_kernel(L, t_chunk, B_blk, H, T, T_pad, unroll)

    h_n, c_n, logits = pl.pallas_call(
        body,
        out_shape=out_shape,
        grid_spec=pltpu.PrefetchScalarGridSpec(
            num_scalar_prefetch=0,
            grid=(n_bblk, n_chunks),
            in_specs=in_specs,
            out_specs=out_specs,
            scratch_shapes=scratch_shapes,
        ),
        compiler_params=pltpu.CompilerParams(
            dimension_semantics=("parallel", "arbitrary"),
            vmem_limit_bytes=60 * 1024 * 1024,
        ),
    )(*flat_inputs)

    return logits[:B, :V], (h_n[:, :B, :], c_n[:, :B, :])


def kernel(embedding, wih_t_0, wih_t_1, whh_t_0, whh_t_1, bias_0, bias_1,
           fc_w_t, fc_b, input_sequence, state_h, state_c):
    args = (embedding, [wih_t_0, wih_t_1], [whh_t_0, whh_t_1],
            [bias_0, bias_1], fc_w_t, fc_b, input_sequence, state_h, state_c)
    try:
        out = _impl(*args, t_chunk=32, unroll=8, single_buffered=True)
        jax.block_until_ready(out)
        return out
    except Exception:
        return _impl(*args, t_chunk=16, unroll=8, single_buffered=False)


# speedup vs baseline: 1.3969x; 1.3969x over previous
import jax
import jax.numpy as jnp
from jax import lax
from jax.experimental import pallas as pl
from jax.experimental.pallas import tpu as pltpu


def _round_up(x, m):
    return (x + m - 1) // m * m


def _make_kernel(t_chunk, n_chunks, b, hidden, unroll):
    B, H = b, hidden

    def _gates(z, c_old):
        sig = jax.nn.sigmoid(z[:, 0:3 * H])
        i_g = sig[:, 0 * H:1 * H]
        f_g = sig[:, 1 * H:2 * H]
        o_g = sig[:, 2 * H:3 * H]
        g_g = jnp.tanh(z[:, 3 * H:4 * H])
        c_new = f_g * c_old + i_g * g_g
        h_new = o_g * jnp.tanh(c_new)
        return h_new, c_new

    def _body(x_ref, w0_ref, b0_ref, w1_ref, b1_ref, fcw_ref, fcb_ref,
              h0_ref, c0_ref, hN_ref, cN_ref, out_ref, hc_scr, y_scr):
        c_idx = pl.program_id(0)

        @pl.when(c_idx == 0)
        def _init():
            hc_scr[0] = h0_ref[0]
            hc_scr[1] = c0_ref[0]
            hc_scr[2] = h0_ref[1]
            hc_scr[3] = c0_ref[1]

        def step(i, carry):
            h0, c0, h1, c1, y_prev = carry
            row = pl.multiple_of(i * B, 8)
            a0 = jnp.concatenate(
                [x_ref[pl.ds(row, B), :], h0.astype(jnp.bfloat16)], axis=1)
            z0 = jnp.dot(a0, w0_ref[...],
                         preferred_element_type=jnp.float32) + b0_ref[...]
            a1 = jnp.concatenate([y_prev, h1.astype(jnp.bfloat16)], axis=1)
            z1 = jnp.dot(a1, w1_ref[...],
                         preferred_element_type=jnp.float32) + b1_ref[...]
            h0_n, c0_n = _gates(z0, c0)
            h1_n, c1_n = _gates(z1, c1)
            l1_valid = (c_idx * t_chunk + i) >= 1
            h1_n = jnp.where(l1_valid, h1_n, h1)
            c1_n = jnp.where(l1_valid, c1_n, c1)
            return (h0_n, c0_n, h1_n, c1_n, h0_n.astype(jnp.bfloat16))

        init = (hc_scr[0], hc_scr[1], hc_scr[2], hc_scr[3], y_scr[...])
        h0, c0, h1, c1, y_prev = lax.fori_loop(0, t_chunk, step, init,
                                               unroll=unroll)
        hc_scr[0] = h0
        hc_scr[1] = c0
        hc_scr[2] = h1
        hc_scr[3] = c1
        y_scr[...] = y_prev

        @pl.when(c_idx == n_chunks - 1)
        def _finalize():
            a1 = jnp.concatenate([y_prev, h1.astype(jnp.bfloat16)], axis=1)
            z1 = jnp.dot(a1, w1_ref[...],
                         preferred_element_type=jnp.float32) + b1_ref[...]
            h1_f, c1_f = _gates(z1, c1)
            hc_scr[2] = h1_f
            hc_scr[3] = c1_f
            hN_ref[0] = h0
            hN_ref[1] = h1_f
            cN_ref[0] = c0
            cN_ref[1] = c1_f
            out_ref[...] = (jnp.dot(h1_f.astype(jnp.bfloat16), fcw_ref[...],
                                    preferred_element_type=jnp.float32)
                            + fcb_ref[...])

    return _body


def kernel(embedding, wih_t_0, wih_t_1, whh_t_0, whh_t_1, bias_0, bias_1,
           fc_w_t, fc_b, input_sequence, state_h, state_c):
    B, T = input_sequence.shape
    E = embedding.shape[1]
    H = state_h.shape[-1]
    V = fc_w_t.shape[1]

    for cand in (32, 16, 8, 4, 2, 1):
        if T % cand == 0:
            t_chunk = cand
            break
    n_chunks = T // t_chunk

    def reorder(w):
        return jnp.concatenate(
            [w[:, :2 * H], w[:, 3 * H:4 * H], w[:, 2 * H:3 * H]], axis=1)

    w0 = jnp.concatenate([reorder(wih_t_0), reorder(whh_t_0)],
                         axis=0).astype(jnp.bfloat16)
    w1 = jnp.concatenate([reorder(wih_t_1), reorder(whh_t_1)],
                         axis=0).astype(jnp.bfloat16)
    b0 = reorder(bias_0).astype(jnp.float32)
    b1 = reorder(bias_1).astype(jnp.float32)

    V_pad = _round_up(V, 128)
    fcw = fc_w_t
    fcb = fc_b
    if V_pad != V:
        fcw = jnp.pad(fcw, ((0, 0), (0, V_pad - V)))
        fcb = jnp.pad(fcb, ((0, 0), (0, V_pad - V)))
    fcw = fcw.astype(jnp.bfloat16)
    fcb = fcb.astype(jnp.float32)

    tok = input_sequence.T.reshape(-1)
    x2d = jnp.take(embedding.astype(jnp.bfloat16), tok, axis=0)

    body = _make_kernel(t_chunk, n_chunks, B, H, unroll=4)

    in_specs = [
        pl.BlockSpec((t_chunk * B, E), lambda c: (c, 0)),
        pl.BlockSpec(w0.shape, lambda c: (0, 0)),
        pl.BlockSpec(b0.shape, lambda c: (0, 0)),
        pl.BlockSpec(w1.shape, lambda c: (0, 0)),
        pl.BlockSpec(b1.shape, lambda c: (0, 0)),
        pl.BlockSpec(fcw.shape, lambda c: (0, 0)),
        pl.BlockSpec(fcb.shape, lambda c: (0, 0)),
        pl.BlockSpec((2, B, H), lambda c: (0, 0, 0)),
        pl.BlockSpec((2, B, H), lambda c: (0, 0, 0)),
    ]
    out_shape = (jax.ShapeDtypeStruct((2, B, H), jnp.float32),
                 jax.ShapeDtypeStruct((2, B, H), jnp.float32),
                 jax.ShapeDtypeStruct((B, V_pad), jnp.float32))
    out_specs = (pl.BlockSpec((2, B, H), lambda c: (0, 0, 0)),
                 pl.BlockSpec((2, B, H), lambda c: (0, 0, 0)),
                 pl.BlockSpec((B, V_pad), lambda c: (0, 0)))
    scratch_shapes = [
        pltpu.VMEM((4, B, H), jnp.float32),
        pltpu.VMEM((B, H), jnp.bfloat16),
    ]

    h_n, c_n, logits = pl.pallas_call(
        body,
        out_shape=out_shape,
        grid_spec=pltpu.PrefetchScalarGridSpec(
            num_scalar_prefetch=0,
            grid=(n_chunks,),
            in_specs=in_specs,
            out_specs=out_specs,
            scratch_shapes=scratch_shapes,
        ),
        compiler_params=pltpu.CompilerParams(
            dimension_semantics=("arbitrary",),
            vmem_limit_bytes=60 * 1024 * 1024,
        ),
    )(x2d, w0, b0, w1, b1, fcw, fcb,
      state_h.astype(jnp.float32), state_c.astype(jnp.float32))

    return logits[:, :V], (h_n, c_n)
